# baseline (speedup 1.0000x reference)
import functools

import jax
import jax.numpy as jnp
import numpy as np
from jax.experimental import pallas as pl

_TINY = np.float32(np.finfo(np.float32).tiny)
_KS0 = np.uint32(0)
_KS1 = np.uint32(42)
_KS2 = np.uint32(0x1BD11BDA) ^ _KS1


def _rotl(x, d):
    return (x << np.uint32(d)) | (x >> np.uint32(32 - d))


def _threefry_bits(i0):
    x0 = jnp.full_like(i0, _KS0)
    x1 = i0 + _KS1

    def rounds(x0, x1, rots):
        for r in rots:
            x0 = x0 + x1
            x1 = _rotl(x1, r)
            x1 = x0 ^ x1
        return x0, x1

    ra = (13, 15, 26, 6)
    rb = (17, 29, 16, 24)
    x0, x1 = rounds(x0, x1, ra)
    x0, x1 = x0 + _KS1, x1 + (_KS2 + np.uint32(1))
    x0, x1 = rounds(x0, x1, rb)
    x0, x1 = x0 + _KS2, x1 + (_KS0 + np.uint32(2))
    x0, x1 = rounds(x0, x1, ra)
    x0, x1 = x0 + _KS0, x1 + (_KS1 + np.uint32(3))
    x0, x1 = rounds(x0, x1, rb)
    x0, x1 = x0 + _KS1, x1 + (_KS2 + np.uint32(4))
    x0, x1 = rounds(x0, x1, ra)
    x0, x1 = x0 + _KS2, x1 + (_KS0 + np.uint32(5))
    return x0 ^ x1


def _row_kernel(x_ref, o_ref, *, sub, chunk, vocab):
    r = pl.program_id(0)
    x = x_ref[0]

    srow = jax.lax.broadcasted_iota(jnp.uint32, (sub, chunk), 0)
    col = jax.lax.broadcasted_iota(jnp.uint32, (sub, chunk), 1)
    v_idx = srow * np.uint32(chunk) + col
    i0 = jnp.uint32(r) * np.uint32(vocab) + v_idx

    bits = _threefry_bits(i0)
    float_bits = (bits >> np.uint32(9)) | np.uint32(0x3F800000)
    u0 = jax.lax.bitcast_convert_type(float_bits, jnp.float32) - np.float32(1.0)
    u = jnp.maximum(_TINY, u0 * (np.float32(1.0) - _TINY) + _TINY)
    g = -jnp.log(-jnp.log(u))
    y = g + x

    m = jnp.max(y)
    big = np.int32(vocab)
    cand = jnp.where(y == m, v_idx.astype(jnp.int32), big)
    a = jnp.min(cand)
    o_ref[0] = (v_idx.astype(jnp.int32) == a).astype(jnp.float32)


def kernel(x):
    m, n, vocab = x.shape
    rows = m * n
    sub = 8
    assert vocab % sub == 0
    chunk = vocab // sub
    xr = x.reshape(rows, sub, chunk)
    out = pl.pallas_call(
        functools.partial(_row_kernel, sub=sub, chunk=chunk, vocab=vocab),
        grid=(rows,),
        in_specs=[pl.BlockSpec((1, sub, chunk), lambda r: (r, 0, 0))],
        out_specs=pl.BlockSpec((1, sub, chunk), lambda r: (r, 0, 0)),
        out_shape=jax.ShapeDtypeStruct((rows, sub, chunk), jnp.float32),
    )(xr)
    return out.reshape(m, n, vocab)
